# baseline (speedup 1.0000x reference)
import functools
import math

import jax
import jax.numpy as jnp
from jax import lax
from jax.experimental import pallas as pl
from jax.experimental.pallas import tpu as pltpu
from jax.experimental.pallas import tpu_sc as plsc

NC = 2
NS = 16
NW = NC * NS
CHUNK = 96
NBUF = 2
LANES = 16


def _spmm_body(nch, nb, n, d, zb, with_deg, refs):
    if with_deg:
        (x_hbm, col_hbm, row_hbm, val_hbm, out_hbm, deg_hbm,
         colv, rowv, valv, rows, zbuf, zdeg, acc, accd,
         gsem, ssem, dsem) = refs
    else:
        (x_hbm, col_hbm, row_hbm, val_hbm, out_hbm,
         colv, rowv, valv, rows, zbuf, acc, gsem, ssem) = refs
    c = lax.axis_index("c")
    s = lax.axis_index("s")
    w = s * NC + c
    npt = ((n // NS + zb - 1) // zb) * zb
    nzc = npt // zb

    pltpu.sync_copy(col_hbm.at[w], colv)
    pltpu.sync_copy(row_hbm.at[w], rowv)
    pltpu.sync_copy(val_hbm.at[w], valv)

    zvec = jnp.zeros((LANES,), jnp.float32)

    def zbody(i, carry):
        for j in range(d // LANES):
            zbuf[i, pl.ds(j * LANES, LANES)] = zvec
        return carry

    lax.fori_loop(0, zb, zbody, 0)

    if with_deg:
        def zdbody(i, carry):
            zdeg[pl.ds(i * LANES, LANES)] = zvec
            return carry

        lax.fori_loop(0, npt // LANES, zdbody, 0)
        pltpu.sync_copy(zdeg, accd.at[pl.ds(s * npt, npt)])

    ngr = nch // NBUF

    def batch_body(b, bcarry):
        for k in range(nzc):
            off = s * npt + k * zb

            @pl.when(off < n)
            def _():
                pltpu.sync_copy(zbuf, acc.at[pl.ds(off, zb)])
        plsc.subcore_barrier()

        def group_body(gi, carry):
            gd = []
            for k in range(NBUF):
                ci = gi * NBUF + k
                gd.append(pltpu.async_copy(
                    x_hbm.at[b].at[colv.at[ci]], rows.at[k], gsem.at[k]))
            sd = []
            for k in range(NBUF):
                ci = gi * NBUF + k
                gd[k].wait()

                def edge_body(g, ecarry):
                    vals16 = valv[ci, pl.ds(g * LANES, LANES)]
                    for i in range(LANES):
                        e = g * LANES + i
                        vv = jnp.full((LANES,), vals16[i], jnp.float32)
                        for j in range(d // LANES):
                            sl = pl.ds(j * LANES, LANES)
                            rows[k, e, sl] = rows[k, e, sl] * vv
                    return ecarry

                lax.fori_loop(0, CHUNK // LANES, edge_body, 0)
                sd.append(pltpu.async_copy(
                    rows.at[k], acc.at[rowv.at[ci]], ssem.at[k], add=True))
                if with_deg:
                    @pl.when(b == 0)
                    def _():
                        pltpu.async_copy(valv.at[ci], accd.at[rowv.at[ci]],
                                         dsem, add=True).wait()
            for k in range(NBUF):
                sd[k].wait()
            return carry

        lax.fori_loop(0, ngr, group_body, 0)
        plsc.subcore_barrier()

        for k in range(nzc):
            off = s * npt + k * zb

            @pl.when(off < n)
            def _():
                sl = pl.ds(off, zb)
                pltpu.sync_copy(acc.at[sl], out_hbm.at[c, b, sl])
        plsc.subcore_barrier()
        return bcarry

    lax.fori_loop(0, nb, batch_body, 0)

    if with_deg:
        sl = pl.ds(s * npt, npt)
        pltpu.sync_copy(accd.at[sl], deg_hbm.at[c, sl])
        plsc.subcore_barrier()


def _spmm(x, col3, row3, val3, with_deg=False):
    nb, n, d = x.shape
    nch = col3.shape[1]
    zb = 16 if n % 16 == 0 else 8
    npt = ((n // NS + zb - 1) // zb) * zb
    mesh = plsc.VectorSubcoreMesh(core_axis_name="c", subcore_axis_name="s",
                                  num_cores=NC, num_subcores=NS)

    def body(*refs):
        _spmm_body(nch, nb, n, d, zb, with_deg, refs)

    out_type = [jax.ShapeDtypeStruct((NC, nb, n, d), jnp.float32)]
    scratch = [
        pltpu.VMEM((nch, CHUNK), jnp.int32),
        pltpu.VMEM((nch, CHUNK), jnp.int32),
        pltpu.VMEM((nch, CHUNK), jnp.float32),
        pltpu.VMEM((NBUF, CHUNK, d), jnp.float32),
        pltpu.VMEM((zb, d), jnp.float32),
        pltpu.VMEM_SHARED((n, d), jnp.float32),
        pltpu.SemaphoreType.DMA((NBUF,)),
        pltpu.SemaphoreType.DMA((NBUF,)),
    ]
    if with_deg:
        out_type.append(jax.ShapeDtypeStruct((NC, NS * npt), jnp.float32))
        scratch.insert(5, pltpu.VMEM((npt,), jnp.float32))
        scratch.insert(7, pltpu.VMEM_SHARED((NS * npt,), jnp.float32))
        scratch.append(pltpu.SemaphoreType.DMA)
    res = pl.kernel(
        body,
        out_type=out_type,
        mesh=mesh,
        scratch_types=scratch,
    )(x, col3, row3, val3)
    return res if with_deg else res[0]


_BN = 1000


def _layer0_body(z_ref, wi_ref, bi_ref, pe_ref, ap_ref, dg_ref,
                 w_ref, bl_ref, g_ref, be_ref, out_ref):
    h0 = jnp.dot(z_ref[0], wi_ref[...], preferred_element_type=jnp.float32)
    h0 = h0 + bi_ref[...]
    ape = ap_ref[0, 0] + ap_ref[1, 0]
    deg = dg_ref[0] + dg_ref[1]
    y = deg * h0 + ape
    t = jnp.dot(y, w_ref[...], preferred_element_type=jnp.float32) + bl_ref[...]
    mu = jnp.mean(t, axis=-1, keepdims=True)
    var = jnp.mean(jnp.square(t - mu), axis=-1, keepdims=True)
    t = (t - mu) * lax.rsqrt(var + 1e-5) * g_ref[...] + be_ref[...]
    ge = 0.5 * t * (1.0 + lax.erf(t * (1.0 / math.sqrt(2.0))))
    out_ref[...] = (pe_ref[...] + h0 + ge)[None]


def _dense_body(yp_ref, h_ref, w_ref, bl_ref, g_ref, be_ref, out_ref):
    y = yp_ref[0, 0] + yp_ref[1, 0]
    t = jnp.dot(y, w_ref[...], preferred_element_type=jnp.float32) + bl_ref[...]
    mu = jnp.mean(t, axis=-1, keepdims=True)
    var = jnp.mean(jnp.square(t - mu), axis=-1, keepdims=True)
    t = (t - mu) * lax.rsqrt(var + 1e-5) * g_ref[...] + be_ref[...]
    ge = 0.5 * t * (1.0 + lax.erf(t * (1.0 / math.sqrt(2.0))))
    out_ref[...] = h_ref[...] + ge[None]


def _heads_body(h_ref, w_ref, b_ref, out_ref):
    t = jnp.dot(h_ref[0], w_ref[...], preferred_element_type=jnp.float32)
    t = t + b_ref[...]
    col = lax.broadcasted_iota(jnp.int32, t.shape, 1)
    out_ref[...] = jnp.where(col < 2, jax.nn.sigmoid(t), t)[None]


def _layer0(zp, w_inp, bin2, pe, ap, dg3, w0, b02, g02, be02):
    zp = zp[:, None, :]
    nb = zp.shape[0]
    n, d = pe.shape
    grid = (nb, n // _BN)
    return pl.pallas_call(
        _layer0_body,
        grid=grid,
        in_specs=[
            pl.BlockSpec((1, 1, zp.shape[2]), lambda b, i: (b, 0, 0)),
            pl.BlockSpec(w_inp.shape, lambda b, i: (0, 0)),
            pl.BlockSpec(bin2.shape, lambda b, i: (0, 0)),
            pl.BlockSpec((_BN, d), lambda b, i: (i, 0)),
            pl.BlockSpec((NC, 1, _BN, d), lambda b, i: (0, 0, i, 0)),
            pl.BlockSpec((NC, _BN, 1), lambda b, i: (0, i, 0)),
            pl.BlockSpec((d, d), lambda b, i: (0, 0)),
            pl.BlockSpec((1, d), lambda b, i: (0, 0)),
            pl.BlockSpec((1, d), lambda b, i: (0, 0)),
            pl.BlockSpec((1, d), lambda b, i: (0, 0)),
        ],
        out_specs=pl.BlockSpec((1, _BN, d), lambda b, i: (b, i, 0)),
        out_shape=jax.ShapeDtypeStruct((nb, n, d), jnp.float32),
    )(zp, w_inp, bin2, pe, ap, dg3, w0, b02, g02, be02)


def _dense(yp, h, wl, bl2, g2, be2):
    nb, n, d = h.shape
    grid = (nb, n // _BN)
    return pl.pallas_call(
        _dense_body,
        grid=grid,
        in_specs=[
            pl.BlockSpec((NC, 1, _BN, d), lambda b, i: (0, b, i, 0)),
            pl.BlockSpec((1, _BN, d), lambda b, i: (b, i, 0)),
            pl.BlockSpec((d, d), lambda b, i: (0, 0)),
            pl.BlockSpec((1, d), lambda b, i: (0, 0)),
            pl.BlockSpec((1, d), lambda b, i: (0, 0)),
            pl.BlockSpec((1, d), lambda b, i: (0, 0)),
        ],
        out_specs=pl.BlockSpec((1, _BN, d), lambda b, i: (b, i, 0)),
        out_shape=jax.ShapeDtypeStruct((nb, n, d), jnp.float32),
    )(yp, h, wl, bl2, g2, be2)


def _heads(h, w_all, b_all2):
    nb, n, d = h.shape
    no = w_all.shape[1]
    grid = (nb, n // _BN)
    return pl.pallas_call(
        _heads_body,
        grid=grid,
        in_specs=[
            pl.BlockSpec((1, _BN, d), lambda b, i: (b, i, 0)),
            pl.BlockSpec((d, no), lambda b, i: (0, 0)),
            pl.BlockSpec((1, no), lambda b, i: (0, 0)),
        ],
        out_specs=pl.BlockSpec((1, _BN, no), lambda b, i: (b, i, 0)),
        out_shape=jax.ShapeDtypeStruct((nb, n, no), jnp.float32),
    )(h, w_all, b_all2)


def kernel(cond_vec, noise, row, col, A_vals, pe, W_in, b_in, gcn_W, gcn_b,
           gcn_g, gcn_beta, W_int, b_int, W_road, b_road, W_zone, b_zone):
    nb = cond_vec.shape[0]
    n, d = pe.shape
    e = row.shape[0]
    nl = gcn_W.shape[0]
    nz = W_zone.shape[1]
    gh = gw = int(math.isqrt(n))

    gsz = CHUNK * NBUF
    ept = ((e + NW * gsz - 1) // (NW * gsz)) * gsz
    padn = NW * ept - e
    col_p = jnp.concatenate([col, jnp.zeros((padn,), jnp.int32)])
    row_p = jnp.concatenate([row, jnp.zeros((padn,), jnp.int32)])
    val_p = jnp.concatenate([A_vals, jnp.zeros((padn,), jnp.float32)])
    nch = ept // CHUNK
    col3 = col_p.reshape(NW, nch, CHUNK)
    row3 = row_p.reshape(NW, nch, CHUNK)
    val3 = val_p.reshape(NW, nch, CHUNK)

    z = jnp.concatenate([noise, cond_vec], axis=-1)
    kp = ((z.shape[1] + 7) // 8) * 8
    zp = jnp.pad(z, ((0, 0), (0, kp - z.shape[1])))
    w_inp = jnp.pad(W_in, ((0, kp - W_in.shape[0]), (0, 0)))

    no = ((2 + nz + 7) // 8) * 8
    w_all = jnp.concatenate(
        [W_int, W_road, W_zone, jnp.zeros((d, no - 2 - nz), jnp.float32)], axis=1)
    b_all = jnp.concatenate(
        [b_int, b_road, b_zone, jnp.zeros((no - 2 - nz,), jnp.float32)])

    ap, dg = _spmm(pe[None], col3, row3, val3, with_deg=True)
    h = _layer0(zp, w_inp, b_in[None], pe, ap, dg[:, :n, None],
                gcn_W[0], gcn_b[0][None], gcn_g[0][None], gcn_beta[0][None])

    for l in range(1, nl):
        yp = _spmm(h, col3, row3, val3)
        h = _dense(yp, h, gcn_W[l], gcn_b[l][None], gcn_g[l][None],
                   gcn_beta[l][None])
    out16 = _heads(h, w_all, b_all[None])

    out_int = out16[..., 0:1].transpose(0, 2, 1).reshape(nb, 1, gh, gw)
    out_road = out16[..., 1:2].transpose(0, 2, 1).reshape(nb, 1, gh, gw)
    zone = out16[..., 2:2 + nz].transpose(0, 2, 1).reshape(nb, nz, gh, gw)
    return (out_int, out_road, zone)


# speedup vs baseline: 1.2447x; 1.2447x over previous
import functools
import math

import jax
import jax.numpy as jnp
from jax import lax
from jax.experimental import pallas as pl
from jax.experimental.pallas import tpu as pltpu
from jax.experimental.pallas import tpu_sc as plsc

NC = 2
NS = 16
NW = NC * NS
CHUNK = 64
NBUF = 2
LANES = 16


def _spmm_body(nch, nb, n, d, zb, with_deg, refs):
    if with_deg:
        (x_hbm, col_hbm, row_hbm, val_hbm, out_hbm, deg_hbm,
         colv, rowv, valv, rows, zbuf, zdeg, acc, accd,
         gsem, ssem, osem, dsem) = refs
    else:
        (x_hbm, col_hbm, row_hbm, val_hbm, out_hbm,
         colv, rowv, valv, rows, zbuf, acc, gsem, ssem, osem) = refs
    c = lax.axis_index("c")
    s = lax.axis_index("s")
    w = s * NC + c
    npt = ((n // NS + zb - 1) // zb) * zb
    nzc = npt // zb

    pltpu.sync_copy(col_hbm.at[w], colv)
    pltpu.sync_copy(row_hbm.at[w], rowv)
    pltpu.sync_copy(val_hbm.at[w], valv)

    zvec = jnp.zeros((LANES,), jnp.float32)

    def zbody(i, carry):
        for j in range(d // LANES):
            zbuf[i, pl.ds(j * LANES, LANES)] = zvec
        return carry

    lax.fori_loop(0, zb, zbody, 0)

    if with_deg:
        def zdbody(i, carry):
            zdeg[pl.ds(i * LANES, LANES)] = zvec
            return carry

        lax.fori_loop(0, npt // LANES, zdbody, 0)
        pltpu.sync_copy(zdeg, accd.at[pl.ds(s * npt, npt)])

    ngr = nch // NBUF

    def batch_body(b, bcarry):
        def zero_issue(k, zcarry):
            off = pl.multiple_of(jnp.minimum(s * npt + k * zb, n - zb), zb)
            pltpu.async_copy(zbuf, acc.at[pl.ds(off, zb)], osem)
            return zcarry

        def zero_drain(k, zcarry):
            off = pl.multiple_of(jnp.minimum(s * npt + k * zb, n - zb), zb)
            pltpu.make_async_copy(zbuf, acc.at[pl.ds(off, zb)], osem).wait()
            return zcarry

        lax.fori_loop(0, nzc, zero_issue, 0)
        lax.fori_loop(0, nzc, zero_drain, 0)
        plsc.subcore_barrier()

        def group_body(gi, carry):
            gd = []
            for k in range(NBUF):
                ci = gi * NBUF + k
                gd.append(pltpu.async_copy(
                    x_hbm.at[b].at[colv.at[ci]], rows.at[k], gsem.at[k]))
            sd = []
            for k in range(NBUF):
                ci = gi * NBUF + k
                gd[k].wait()

                def edge_body(g, ecarry):
                    vals16 = valv[ci, pl.ds(g * LANES, LANES)]
                    for i in range(LANES):
                        e = g * LANES + i
                        vv = jnp.full((LANES,), vals16[i], jnp.float32)
                        for j in range(d // LANES):
                            sl = pl.ds(j * LANES, LANES)
                            rows[k, e, sl] = rows[k, e, sl] * vv
                    return ecarry

                lax.fori_loop(0, CHUNK // LANES, edge_body, 0)
                sd.append(pltpu.async_copy(
                    rows.at[k], acc.at[rowv.at[ci]], ssem.at[k], add=True))
                if with_deg:
                    @pl.when(b == 0)
                    def _():
                        pltpu.async_copy(valv.at[ci], accd.at[rowv.at[ci]],
                                         dsem, add=True).wait()
            for k in range(NBUF):
                sd[k].wait()
            return carry

        lax.fori_loop(0, ngr, group_body, 0)
        plsc.subcore_barrier()

        def out_issue(k, ocarry):
            off = pl.multiple_of(jnp.minimum(s * npt + k * zb, n - zb), zb)
            sl = pl.ds(off, zb)
            pltpu.async_copy(acc.at[sl], out_hbm.at[c, b, sl], osem)
            return ocarry

        def out_drain(k, ocarry):
            off = pl.multiple_of(jnp.minimum(s * npt + k * zb, n - zb), zb)
            sl = pl.ds(off, zb)
            pltpu.make_async_copy(acc.at[sl], out_hbm.at[c, b, sl],
                                  osem).wait()
            return ocarry

        lax.fori_loop(0, nzc, out_issue, 0)
        lax.fori_loop(0, nzc, out_drain, 0)
        plsc.subcore_barrier()
        return bcarry

    lax.fori_loop(0, nb, batch_body, 0)

    if with_deg:
        sl = pl.ds(s * npt, npt)
        pltpu.sync_copy(accd.at[sl], deg_hbm.at[c, sl])
        plsc.subcore_barrier()


def _spmm(x, col3, row3, val3, with_deg=False):
    nb, n, d = x.shape
    nch = col3.shape[1]
    zb = 16 if n % 16 == 0 else 8
    npt = ((n // NS + zb - 1) // zb) * zb
    mesh = plsc.VectorSubcoreMesh(core_axis_name="c", subcore_axis_name="s",
                                  num_cores=NC, num_subcores=NS)

    def body(*refs):
        _spmm_body(nch, nb, n, d, zb, with_deg, refs)

    out_type = [jax.ShapeDtypeStruct((NC, nb, n, d), jnp.float32)]
    scratch = [
        pltpu.VMEM((nch, CHUNK), jnp.int32),
        pltpu.VMEM((nch, CHUNK), jnp.int32),
        pltpu.VMEM((nch, CHUNK), jnp.float32),
        pltpu.VMEM((NBUF, CHUNK, d), jnp.float32),
        pltpu.VMEM((zb, d), jnp.float32),
        pltpu.VMEM_SHARED((n, d), jnp.float32),
        pltpu.SemaphoreType.DMA((NBUF,)),
        pltpu.SemaphoreType.DMA((NBUF,)),
        pltpu.SemaphoreType.DMA,
    ]
    if with_deg:
        out_type.append(jax.ShapeDtypeStruct((NC, NS * npt), jnp.float32))
        scratch.insert(5, pltpu.VMEM((npt,), jnp.float32))
        scratch.insert(7, pltpu.VMEM_SHARED((NS * npt,), jnp.float32))
        scratch.append(pltpu.SemaphoreType.DMA)
    res = pl.kernel(
        body,
        out_type=out_type,
        mesh=mesh,
        scratch_types=scratch,
    )(x, col3, row3, val3)
    return res if with_deg else res[0]


_BN = 1000


def _layer0_body(z_ref, wi_ref, bi_ref, pe_ref, ap_ref, dg_ref,
                 w_ref, bl_ref, g_ref, be_ref, out_ref):
    h0 = jnp.dot(z_ref[0], wi_ref[...], preferred_element_type=jnp.float32)
    h0 = h0 + bi_ref[...]
    ape = ap_ref[0, 0] + ap_ref[1, 0]
    deg = dg_ref[0] + dg_ref[1]
    y = deg * h0 + ape
    t = jnp.dot(y, w_ref[...], preferred_element_type=jnp.float32) + bl_ref[...]
    mu = jnp.mean(t, axis=-1, keepdims=True)
    var = jnp.mean(jnp.square(t - mu), axis=-1, keepdims=True)
    t = (t - mu) * lax.rsqrt(var + 1e-5) * g_ref[...] + be_ref[...]
    ge = 0.5 * t * (1.0 + lax.erf(t * (1.0 / math.sqrt(2.0))))
    out_ref[...] = (pe_ref[...] + h0 + ge)[None]


def _dense_body(yp_ref, h_ref, w_ref, bl_ref, g_ref, be_ref, out_ref):
    y = yp_ref[0, 0] + yp_ref[1, 0]
    t = jnp.dot(y, w_ref[...], preferred_element_type=jnp.float32) + bl_ref[...]
    mu = jnp.mean(t, axis=-1, keepdims=True)
    var = jnp.mean(jnp.square(t - mu), axis=-1, keepdims=True)
    t = (t - mu) * lax.rsqrt(var + 1e-5) * g_ref[...] + be_ref[...]
    ge = 0.5 * t * (1.0 + lax.erf(t * (1.0 / math.sqrt(2.0))))
    out_ref[...] = h_ref[...] + ge[None]


def _heads_body(h_ref, w_ref, b_ref, out_ref):
    t = jnp.dot(h_ref[0], w_ref[...], preferred_element_type=jnp.float32)
    t = t + b_ref[...]
    col = lax.broadcasted_iota(jnp.int32, t.shape, 1)
    out_ref[...] = jnp.where(col < 2, jax.nn.sigmoid(t), t)[None]


def _layer0(zp, w_inp, bin2, pe, ap, dg3, w0, b02, g02, be02):
    zp = zp[:, None, :]
    nb = zp.shape[0]
    n, d = pe.shape
    grid = (nb, n // _BN)
    return pl.pallas_call(
        _layer0_body,
        grid=grid,
        in_specs=[
            pl.BlockSpec((1, 1, zp.shape[2]), lambda b, i: (b, 0, 0)),
            pl.BlockSpec(w_inp.shape, lambda b, i: (0, 0)),
            pl.BlockSpec(bin2.shape, lambda b, i: (0, 0)),
            pl.BlockSpec((_BN, d), lambda b, i: (i, 0)),
            pl.BlockSpec((NC, 1, _BN, d), lambda b, i: (0, 0, i, 0)),
            pl.BlockSpec((NC, _BN, 1), lambda b, i: (0, i, 0)),
            pl.BlockSpec((d, d), lambda b, i: (0, 0)),
            pl.BlockSpec((1, d), lambda b, i: (0, 0)),
            pl.BlockSpec((1, d), lambda b, i: (0, 0)),
            pl.BlockSpec((1, d), lambda b, i: (0, 0)),
        ],
        out_specs=pl.BlockSpec((1, _BN, d), lambda b, i: (b, i, 0)),
        out_shape=jax.ShapeDtypeStruct((nb, n, d), jnp.float32),
    )(zp, w_inp, bin2, pe, ap, dg3, w0, b02, g02, be02)


def _dense(yp, h, wl, bl2, g2, be2):
    nb, n, d = h.shape
    grid = (nb, n // _BN)
    return pl.pallas_call(
        _dense_body,
        grid=grid,
        in_specs=[
            pl.BlockSpec((NC, 1, _BN, d), lambda b, i: (0, b, i, 0)),
            pl.BlockSpec((1, _BN, d), lambda b, i: (b, i, 0)),
            pl.BlockSpec((d, d), lambda b, i: (0, 0)),
            pl.BlockSpec((1, d), lambda b, i: (0, 0)),
            pl.BlockSpec((1, d), lambda b, i: (0, 0)),
            pl.BlockSpec((1, d), lambda b, i: (0, 0)),
        ],
        out_specs=pl.BlockSpec((1, _BN, d), lambda b, i: (b, i, 0)),
        out_shape=jax.ShapeDtypeStruct((nb, n, d), jnp.float32),
    )(yp, h, wl, bl2, g2, be2)


def _heads(h, w_all, b_all2):
    nb, n, d = h.shape
    no = w_all.shape[1]
    grid = (nb, n // _BN)
    return pl.pallas_call(
        _heads_body,
        grid=grid,
        in_specs=[
            pl.BlockSpec((1, _BN, d), lambda b, i: (b, i, 0)),
            pl.BlockSpec((d, no), lambda b, i: (0, 0)),
            pl.BlockSpec((1, no), lambda b, i: (0, 0)),
        ],
        out_specs=pl.BlockSpec((1, _BN, no), lambda b, i: (b, i, 0)),
        out_shape=jax.ShapeDtypeStruct((nb, n, no), jnp.float32),
    )(h, w_all, b_all2)


def kernel(cond_vec, noise, row, col, A_vals, pe, W_in, b_in, gcn_W, gcn_b,
           gcn_g, gcn_beta, W_int, b_int, W_road, b_road, W_zone, b_zone):
    nb = cond_vec.shape[0]
    n, d = pe.shape
    e = row.shape[0]
    nl = gcn_W.shape[0]
    nz = W_zone.shape[1]
    gh = gw = int(math.isqrt(n))

    gsz = CHUNK * NBUF
    ept = ((e + NW * gsz - 1) // (NW * gsz)) * gsz
    padn = NW * ept - e
    col_p = jnp.concatenate([col, jnp.zeros((padn,), jnp.int32)])
    row_p = jnp.concatenate([row, jnp.zeros((padn,), jnp.int32)])
    val_p = jnp.concatenate([A_vals, jnp.zeros((padn,), jnp.float32)])
    nch = ept // CHUNK
    col3 = col_p.reshape(NW, nch, CHUNK)
    row3 = row_p.reshape(NW, nch, CHUNK)
    val3 = val_p.reshape(NW, nch, CHUNK)

    z = jnp.concatenate([noise, cond_vec], axis=-1)
    kp = ((z.shape[1] + 7) // 8) * 8
    zp = jnp.pad(z, ((0, 0), (0, kp - z.shape[1])))
    w_inp = jnp.pad(W_in, ((0, kp - W_in.shape[0]), (0, 0)))

    no = ((2 + nz + 7) // 8) * 8
    w_all = jnp.concatenate(
        [W_int, W_road, W_zone, jnp.zeros((d, no - 2 - nz), jnp.float32)], axis=1)
    b_all = jnp.concatenate(
        [b_int, b_road, b_zone, jnp.zeros((no - 2 - nz,), jnp.float32)])

    ap, dg = _spmm(pe[None], col3, row3, val3, with_deg=True)
    h = _layer0(zp, w_inp, b_in[None], pe, ap, dg[:, :n, None],
                gcn_W[0], gcn_b[0][None], gcn_g[0][None], gcn_beta[0][None])

    for l in range(1, nl):
        yp = _spmm(h, col3, row3, val3)
        h = _dense(yp, h, gcn_W[l], gcn_b[l][None], gcn_g[l][None],
                   gcn_beta[l][None])
    out16 = _heads(h, w_all, b_all[None])

    out_int = out16[..., 0:1].transpose(0, 2, 1).reshape(nb, 1, gh, gw)
    out_road = out16[..., 1:2].transpose(0, 2, 1).reshape(nb, 1, gh, gw)
    zone = out16[..., 2:2 + nz].transpose(0, 2, 1).reshape(nb, nz, gh, gw)
    return (out_int, out_road, zone)
